# speedup vs baseline: 1.4090x; 1.4090x over previous
import functools

import jax
import jax.numpy as jnp
from jax import lax
from jax.experimental import pallas as pl
from jax.experimental.pallas import tpu as pltpu
from jax.experimental.pallas import tpu_sc as plsc

DIM = 128


def _gather3(qw, kw, vw, idx):
    nrows = idx.shape[0]
    info = plsc.get_sparse_core_info()
    nw = info.num_cores * info.num_subcores
    b_per_w = nrows // nw

    mesh = plsc.VectorSubcoreMesh(core_axis_name="c", subcore_axis_name="s")
    out_t = jax.ShapeDtypeStruct((nrows, DIM), jnp.float32)

    @functools.partial(
        pl.kernel,
        mesh=mesh,
        out_type=(out_t, out_t, out_t),
        scratch_types=[
            pltpu.VMEM((b_per_w,), jnp.int32),
            pltpu.VMEM((b_per_w, DIM), jnp.float32),
            pltpu.VMEM((b_per_w, DIM), jnp.float32),
            pltpu.VMEM((b_per_w, DIM), jnp.float32),
            pltpu.SemaphoreType.DMA,
            pltpu.SemaphoreType.DMA,
            pltpu.SemaphoreType.DMA,
        ],
    )
    def k(q_hbm, k_hbm, v_hbm, idx_hbm, oq, ok, ov, idx_v, rq, rk, rv, s0, s1, s2):
        wid = lax.axis_index("s") * info.num_cores + lax.axis_index("c")
        base = wid * b_per_w
        pltpu.sync_copy(idx_hbm.at[pl.ds(base, b_per_w)], idx_v)
        cq = pltpu.async_copy(q_hbm.at[idx_v], rq, s0)
        ck = pltpu.async_copy(k_hbm.at[idx_v], rk, s1)
        cv = pltpu.async_copy(v_hbm.at[idx_v], rv, s2)
        cq.wait()
        pltpu.sync_copy(rq, oq.at[pl.ds(base, b_per_w)])
        ck.wait()
        pltpu.sync_copy(rk, ok.at[pl.ds(base, b_per_w)])
        cv.wait()
        pltpu.sync_copy(rv, ov.at[pl.ds(base, b_per_w)])

    return k(qw, kw, vw, idx)


def kernel(rel_q_weight, rel_k_weight, rel_v_weight, rel_index):
    idx = rel_index.reshape(-1).astype(jnp.int32)
    aq, ak, av = _gather3(rel_q_weight, rel_k_weight, rel_v_weight, idx)
    shp = rel_index.shape + (DIM,)
    return aq.reshape(shp), ak.reshape(shp), av.reshape(shp)


# speedup vs baseline: 1.4123x; 1.0024x over previous
import functools

import jax
import jax.numpy as jnp
from jax import lax
from jax.experimental import pallas as pl
from jax.experimental.pallas import tpu as pltpu
from jax.experimental.pallas import tpu_sc as plsc

DIM = 128


def _gather3(qw, kw, vw, idx):
    nrows = idx.shape[0]
    info = plsc.get_sparse_core_info()
    nw = info.num_cores * info.num_subcores
    b_per_w = nrows // nw

    mesh = plsc.VectorSubcoreMesh(core_axis_name="c", subcore_axis_name="s")
    out_t = jax.ShapeDtypeStruct((nrows, DIM), jnp.float32)

    @functools.partial(
        pl.kernel,
        mesh=mesh,
        out_type=(out_t, out_t, out_t),
        scratch_types=[
            pltpu.VMEM((b_per_w,), jnp.int32),
            pltpu.VMEM((b_per_w, DIM), jnp.float32),
            pltpu.VMEM((b_per_w, DIM), jnp.float32),
            pltpu.VMEM((b_per_w, DIM), jnp.float32),
            pltpu.SemaphoreType.DMA,
            pltpu.SemaphoreType.DMA,
            pltpu.SemaphoreType.DMA,
            pltpu.SemaphoreType.DMA,
            pltpu.SemaphoreType.DMA,
            pltpu.SemaphoreType.DMA,
        ],
    )
    def k(q_hbm, k_hbm, v_hbm, idx_hbm, oq, ok, ov, idx_v, rq, rk, rv,
          s0, s1, s2, t0, t1, t2):
        wid = lax.axis_index("s") * info.num_cores + lax.axis_index("c")
        base = wid * b_per_w
        pltpu.sync_copy(idx_hbm.at[pl.ds(base, b_per_w)], idx_v)
        cq = pltpu.async_copy(q_hbm.at[idx_v], rq, s0)
        ck = pltpu.async_copy(k_hbm.at[idx_v], rk, s1)
        cv = pltpu.async_copy(v_hbm.at[idx_v], rv, s2)
        cq.wait()
        wq = pltpu.async_copy(rq, oq.at[pl.ds(base, b_per_w)], t0)
        ck.wait()
        wk = pltpu.async_copy(rk, ok.at[pl.ds(base, b_per_w)], t1)
        cv.wait()
        wv = pltpu.async_copy(rv, ov.at[pl.ds(base, b_per_w)], t2)
        wq.wait()
        wk.wait()
        wv.wait()

    return k(qw, kw, vw, idx)


def kernel(rel_q_weight, rel_k_weight, rel_v_weight, rel_index):
    idx = rel_index.reshape(-1).astype(jnp.int32)
    aq, ak, av = _gather3(rel_q_weight, rel_k_weight, rel_v_weight, idx)
    shp = rel_index.shape + (DIM,)
    return aq.reshape(shp), ak.reshape(shp), av.reshape(shp)
